# baseline (speedup 1.0000x reference)
import functools

import jax
import jax.numpy as jnp
from jax import lax
from jax.experimental import pallas as pl
from jax.experimental.pallas import tpu as pltpu
from jax.experimental.pallas import tpu_sc as plsc

B = 16384
S = 50
D = 32
L = 16
NC, NS = 2, 16
NW = NC * NS
RW = B // NW
GB = 128
KB = RW // GB


def _make_sc_call():
  mesh = plsc.VectorSubcoreMesh(core_axis_name="c", subcore_axis_name="s")

  @functools.partial(
      pl.kernel,
      out_type=jax.ShapeDtypeStruct((B, D), jnp.float32),
      mesh=mesh,
      compiler_params=pltpu.CompilerParams(needs_layout_passes=False,
                                           use_tc_tiling_on_sc=False),
      scratch_types=[
          pltpu.VMEM((RW * S,), jnp.int32),
          pltpu.VMEM((S, KB, GB), jnp.int32),
          pltpu.VMEM((RW, D), jnp.float32),
          pltpu.VMEM((RW,), jnp.float32),
          pltpu.VMEM((RW,), jnp.float32),
          pltpu.VMEM((1, D), jnp.float32),
          pltpu.SemaphoreType.DMA,
      ],
  )
  def sc_kernel(idx_hbm, emb_hbm, out_hbm,
                idxr_v, idxt_v, acc_v, inv_v, nz_v, e0_v, sem):
    wid = lax.axis_index("s") * NC + lax.axis_index("c")
    row0 = pl.multiple_of(wid * RW, RW)

    pltpu.sync_copy(emb_hbm.at[pl.ds(0, 1)], e0_v)
    pltpu.sync_copy(idx_hbm.at[pl.ds(pl.multiple_of(wid * RW * S, RW * S),
                                     RW * S)], idxr_v)
    lanes = lax.iota(jnp.int32, L)
    lanes_s = lanes * S

    for k in range(KB):
      for l in range(GB // L):
        rbase = (k * GB + l * L) * S
        idxt_v[0, k, pl.ds(l * L, L)] = plsc.load_gather(
            idxr_v, [lanes_s + rbase])
    d0 = [pltpu.async_copy(emb_hbm.at[idxt_v.at[0, k]],
                           acc_v.at[pl.ds(k * GB, GB)], sem)
          for k in range(KB)]

    for k in range(KB):
      for l in range(GB // L):
        gbase = k * GB + l * L
        rbase = gbase * S

        def body(s, cnt, k=k, l=l, rbase=rbase):
          vals = plsc.load_gather(idxr_v, [lanes_s + (rbase + s)])
          idxt_v[s, k, pl.ds(l * L, L)] = vals
          return cnt + (vals != 0).astype(jnp.int32)

        cnt0 = (idxt_v[0, k, pl.ds(l * L, L)] != 0).astype(jnp.int32)
        cnt = lax.fori_loop(1, S, body, cnt0)
        cntf = cnt.astype(jnp.float32)
        inv_v[pl.ds(gbase, L)] = jnp.where(cnt == 0, 0.0,
                                           1.0 / (cntf + 1e-8))
        nz_v[pl.ds(gbase, L)] = jnp.float32(S) - cntf

    for dd in d0:
      dd.wait()

    def fire(s, carry):
      for k in range(KB):
        pltpu.async_copy(emb_hbm.at[idxt_v.at[s, k]],
                         acc_v.at[pl.ds(k * GB, GB)], sem, add=True)
      return carry

    lax.fori_loop(1, S, fire, 0)

    def drain(i, carry):
      pltpu.make_async_copy(emb_hbm.at[idxt_v.at[0, 0]],
                            acc_v.at[pl.ds(0, GB)], sem).wait()
      return carry

    lax.fori_loop(0, (S - 1) * KB, drain, 0)

    e00 = e0_v[0, 0:L]
    e01 = e0_v[0, L:D]

    def row_body(r, carry):
      isplat = jnp.full((L,), r, jnp.int32)
      nz = plsc.load_gather(nz_v, [isplat])
      inv = plsc.load_gather(inv_v, [isplat])
      acc_v[r, 0:L] = (acc_v[r, 0:L] - nz * e00) * inv
      acc_v[r, L:D] = (acc_v[r, L:D] - nz * e01) * inv
      return carry

    lax.fori_loop(0, RW, row_body, 0)

    pltpu.sync_copy(acc_v, out_hbm.at[pl.ds(row0, RW)])

  return sc_kernel


_make_sc_call = functools.cache(_make_sc_call)


def kernel(indices, embeddings):
  return _make_sc_call()(indices.astype(jnp.int32).reshape(B * S), embeddings)


# speedup vs baseline: 1.0111x; 1.0111x over previous
import functools

import jax
import jax.numpy as jnp
from jax import lax
from jax.experimental import pallas as pl
from jax.experimental.pallas import tpu as pltpu
from jax.experimental.pallas import tpu_sc as plsc

B = 16384
S = 50
D = 32
L = 16
NC, NS = 2, 16
NW = NC * NS
RW = B // NW
GB = 128
KB = RW // GB


def _make_sc_call():
  mesh = plsc.VectorSubcoreMesh(core_axis_name="c", subcore_axis_name="s")

  @functools.partial(
      pl.kernel,
      out_type=jax.ShapeDtypeStruct((B, 128), jnp.float32),
      mesh=mesh,
      compiler_params=pltpu.CompilerParams(needs_layout_passes=False,
                                           use_tc_tiling_on_sc=False),
      scratch_types=[
          pltpu.VMEM((RW * S,), jnp.int32),
          pltpu.VMEM((S, KB, GB), jnp.int32),
          pltpu.VMEM((RW, D), jnp.float32),
          pltpu.VMEM((RW,), jnp.float32),
          pltpu.VMEM((RW,), jnp.float32),
          pltpu.VMEM((1, D), jnp.float32),
          pltpu.SemaphoreType.DMA,
      ],
  )
  def sc_kernel(idx_hbm, emb_hbm, out_hbm,
                idxr_v, idxt_v, acc_v, inv_v, nz_v, e0_v, sem):
    wid = lax.axis_index("s") * NC + lax.axis_index("c")
    row0 = pl.multiple_of(wid * RW, RW)

    pltpu.sync_copy(emb_hbm.at[pl.ds(0, 1)], e0_v)
    pltpu.sync_copy(idx_hbm.at[pl.ds(pl.multiple_of(wid * RW * S, RW * S),
                                     RW * S)], idxr_v)
    lanes = lax.iota(jnp.int32, L)
    lanes_s = lanes * S

    for k in range(KB):
      for l in range(GB // L):
        rbase = (k * GB + l * L) * S
        idxt_v[0, k, pl.ds(l * L, L)] = plsc.load_gather(
            idxr_v, [lanes_s + rbase])
    d0 = [pltpu.async_copy(emb_hbm.at[idxt_v.at[0, k]],
                           acc_v.at[pl.ds(k * GB, GB)], sem)
          for k in range(KB)]

    for k in range(KB):
      for l in range(GB // L):
        gbase = k * GB + l * L
        rbase = gbase * S

        def body(s, cnt, k=k, l=l, rbase=rbase):
          vals = plsc.load_gather(idxr_v, [lanes_s + (rbase + s)])
          idxt_v[s, k, pl.ds(l * L, L)] = vals
          return cnt + (vals != 0).astype(jnp.int32)

        cnt0 = (idxt_v[0, k, pl.ds(l * L, L)] != 0).astype(jnp.int32)
        cnt = lax.fori_loop(1, S, body, cnt0)
        cntf = cnt.astype(jnp.float32)
        inv_v[pl.ds(gbase, L)] = jnp.where(cnt == 0, 0.0,
                                           1.0 / (cntf + 1e-8))
        nz_v[pl.ds(gbase, L)] = jnp.float32(S) - cntf

    for dd in d0:
      dd.wait()

    def fire(s, carry):
      for k in range(KB):
        pltpu.async_copy(emb_hbm.at[idxt_v.at[s, k]],
                         acc_v.at[pl.ds(k * GB, GB)], sem, add=True)
      return carry

    lax.fori_loop(1, S, fire, 0)

    def drain(i, carry):
      pltpu.make_async_copy(emb_hbm.at[idxt_v.at[0, 0]],
                            acc_v.at[pl.ds(0, GB)], sem).wait()
      return carry

    lax.fori_loop(0, (S - 1) * KB, drain, 0)

    e00 = e0_v[0, 0:L]
    e01 = e0_v[0, L:D]

    def row_body(r, carry):
      isplat = jnp.full((L,), r, jnp.int32)
      nz = plsc.load_gather(nz_v, [isplat])
      inv = plsc.load_gather(inv_v, [isplat])
      acc_v[r, 0:L] = (acc_v[r, 0:L] - nz * e00) * inv
      acc_v[r, L:D] = (acc_v[r, L:D] - nz * e01) * inv
      return carry

    lax.fori_loop(0, RW, row_body, 0)

    pltpu.sync_copy(acc_v, out_hbm.at[pl.ds(row0, RW), pl.ds(0, D)])

  return sc_kernel


_make_sc_call = functools.cache(_make_sc_call)


def kernel(indices, embeddings):
  padded = _make_sc_call()(indices.astype(jnp.int32).reshape(B * S),
                           embeddings)
  return padded[:, :D]
